# Initial kernel scaffold; baseline (speedup 1.0000x reference)
import jax
import jax.numpy as jnp
from jax.experimental import pallas as pl


def kernel(x, edge_index, edge_attr, batch, Wp, bp, Wm, bm, Wu, bu, W_ih, W_hh, b_ih, b_hh, W1, b1, ln_g, ln_b, W2, b2):
    raise NotImplementedError("write your pallas kernel here")


# speedup vs baseline: 3.8829x; 3.8829x over previous
import functools

import jax
import jax.numpy as jnp
from jax import lax
from jax.experimental import pallas as pl
from jax.experimental.pallas import tpu as pltpu
from jax.experimental.pallas import tpu_sc as plsc

N = 10000
E = 320000
H = 128
EF = 16
L = 3
G = 64
STEPS = 6

_K = 80
_LN = 16


def _celu(v):
    return jnp.where(v > 0, v, jnp.exp(v) - 1.0)


def _proj_ab_body(x_ref, wp_ref, bp_ref, wab_ref, o_ref, a_ref, b_ref):
    h = jnp.dot(x_ref[...], wp_ref[...], preferred_element_type=jnp.float32, precision=lax.Precision.HIGHEST)
    h = _celu(h + bp_ref[...])
    o_ref[...] = h
    ab = jnp.dot(h, wab_ref[...], preferred_element_type=jnp.float32, precision=lax.Precision.HIGHEST)
    a_ref[...] = ab[:, :H]
    b_ref[...] = ab[:, H:]


def _proj_ab(x, Wp, bp2, Wab):
    bn = 2000
    grid = (N // bn,)
    return pl.pallas_call(
        _proj_ab_body,
        grid=grid,
        in_specs=[
            pl.BlockSpec((bn, H), lambda i: (i, 0)),
            pl.BlockSpec((H, H), lambda i: (0, 0)),
            pl.BlockSpec((1, H), lambda i: (0, 0)),
            pl.BlockSpec((H, 2 * H), lambda i: (0, 0)),
        ],
        out_specs=[
            pl.BlockSpec((bn, H), lambda i: (i, 0)),
            pl.BlockSpec((bn, H), lambda i: (i, 0)),
            pl.BlockSpec((bn, H), lambda i: (i, 0)),
        ],
        out_shape=[
            jax.ShapeDtypeStruct((N, H), jnp.float32),
            jax.ShapeDtypeStruct((N, H), jnp.float32),
            jax.ShapeDtypeStruct((N, H), jnp.float32),
        ],
    )(x, Wp, bp2, Wab)


def _edgeproj_body(ea_ref, we_ref, bm_ref, c_ref):
    c_ref[...] = (
        jnp.dot(ea_ref[...], we_ref[...], preferred_element_type=jnp.float32, precision=lax.Precision.HIGHEST)
        + bm_ref[...]
    )


def _edgeproj(edge_attr, We, bm2):
    bn = 4000
    grid = (E // bn,)
    return pl.pallas_call(
        _edgeproj_body,
        grid=grid,
        in_specs=[
            pl.BlockSpec((bn, EF), lambda i: (i, 0)),
            pl.BlockSpec((EF, H), lambda i: (0, 0)),
            pl.BlockSpec((1, H), lambda i: (0, 0)),
        ],
        out_specs=pl.BlockSpec((bn, H), lambda i: (i, 0)),
        out_shape=jax.ShapeDtypeStruct((E, H), jnp.float32),
    )(edge_attr, We, bm2)


def _upd_ab_body(o_ref, p_ref, wu_ref, bu_ref, wab_ref, no_ref, a_ref, b_ref):
    agg = p_ref[0] + p_ref[1]
    o = (
        o_ref[...]
        + jnp.dot(agg, wu_ref[...], preferred_element_type=jnp.float32)
        + bu_ref[...]
    )
    no_ref[...] = o
    ab = jnp.dot(o, wab_ref[...], preferred_element_type=jnp.float32, precision=lax.Precision.HIGHEST)
    a_ref[...] = ab[:, :H]
    b_ref[...] = ab[:, H:]


def _upd_ab(out, P, Wu, bu2, Wab):
    bn = 2000
    grid = (N // bn,)
    return pl.pallas_call(
        _upd_ab_body,
        grid=grid,
        in_specs=[
            pl.BlockSpec((bn, H), lambda i: (i, 0)),
            pl.BlockSpec((2, bn, H), lambda i: (0, i, 0)),
            pl.BlockSpec((H, H), lambda i: (0, 0)),
            pl.BlockSpec((1, H), lambda i: (0, 0)),
            pl.BlockSpec((H, 2 * H), lambda i: (0, 0)),
        ],
        out_specs=[
            pl.BlockSpec((bn, H), lambda i: (i, 0)),
            pl.BlockSpec((bn, H), lambda i: (i, 0)),
            pl.BlockSpec((bn, H), lambda i: (i, 0)),
        ],
        out_shape=[
            jax.ShapeDtypeStruct((N, H), jnp.float32),
            jax.ShapeDtypeStruct((N, H), jnp.float32),
            jax.ShapeDtypeStruct((N, H), jnp.float32),
        ],
    )(out, P, Wu, bu2, Wab)


def _upd_body(o_ref, p_ref, wu_ref, bu_ref, no_ref):
    agg = p_ref[0] + p_ref[1]
    no_ref[...] = (
        o_ref[...]
        + jnp.dot(agg, wu_ref[...], preferred_element_type=jnp.float32)
        + bu_ref[...]
    )


def _upd(out, P, Wu, bu2):
    bn = 2000
    grid = (N // bn,)
    return pl.pallas_call(
        _upd_body,
        grid=grid,
        in_specs=[
            pl.BlockSpec((bn, H), lambda i: (i, 0)),
            pl.BlockSpec((2, bn, H), lambda i: (0, i, 0)),
            pl.BlockSpec((H, H), lambda i: (0, 0)),
            pl.BlockSpec((1, H), lambda i: (0, 0)),
        ],
        out_specs=pl.BlockSpec((bn, H), lambda i: (i, 0)),
        out_shape=jax.ShapeDtypeStruct((N, H), jnp.float32),
    )(out, P, Wu, bu2)


def _s2s_body(out_ref, batch_ref, wih_ref, whh_ref, bg_ref, w1_ref, b1_ref,
              lng_ref, lnb_ref, w2_ref, b2_ref, o_ref):
    out = out_ref[...]
    bvec = batch_ref[...]
    gidx = lax.broadcasted_iota(jnp.int32, (1, G), 1)
    onehot = bvec == gidx
    q_star = jnp.zeros((G, 2 * H), jnp.float32)
    h = jnp.zeros((G, H), jnp.float32)
    c = jnp.zeros((G, H), jnp.float32)
    for _ in range(STEPS):
        gates = (
            jnp.dot(q_star, wih_ref[...], preferred_element_type=jnp.float32)
            + jnp.dot(h, whh_ref[...], preferred_element_type=jnp.float32)
            + bg_ref[...]
        )
        i_g = jax.nn.sigmoid(gates[:, :H])
        f_g = jax.nn.sigmoid(gates[:, H:2 * H])
        g_g = jnp.tanh(gates[:, 2 * H:3 * H])
        o_g = jax.nn.sigmoid(gates[:, 3 * H:])
        c = f_g * c + i_g * g_g
        h = o_g * jnp.tanh(c)
        M = lax.dot_general(out, h, (((1,), (1,)), ((), ())),
                            preferred_element_type=jnp.float32,
                            precision=lax.Precision.HIGHEST)
        seg_max = jnp.max(jnp.where(onehot, M, -3.0e38), axis=0, keepdims=True)
        P = jnp.where(onehot, jnp.exp(M - seg_max), 0.0)
        denom = jnp.sum(P, axis=0, keepdims=True)
        A = P / jnp.maximum(denom, 1e-30)
        r = lax.dot_general(A, out, (((0,), (0,)), ((), ())),
                            preferred_element_type=jnp.float32,
                            precision=lax.Precision.HIGHEST)
        q_star = jnp.concatenate([h, r], axis=1)
    h1 = jnp.dot(q_star, w1_ref[...], preferred_element_type=jnp.float32) + b1_ref[...]
    mu = jnp.mean(h1, axis=-1, keepdims=True)
    var = jnp.mean((h1 - mu) ** 2, axis=-1, keepdims=True)
    h1 = (h1 - mu) / jnp.sqrt(var + 1e-5) * lng_ref[...] + lnb_ref[...]
    h1 = jnp.maximum(h1, 0.0)
    o_ref[...] = jnp.dot(h1, w2_ref[...], preferred_element_type=jnp.float32) + b2_ref[...]


def _s2s(out, batch2d, Wih_t, Whh_t, bg2, W1, b12, lng2, lnb2, W2, b22):
    return pl.pallas_call(
        _s2s_body,
        out_shape=jax.ShapeDtypeStruct((G, H), jnp.float32),
    )(out, batch2d, Wih_t, Whh_t, bg2, W1, b12, lng2, lnb2, W2, b22)


_NC = 2
_NS = 16
_NW = _NC * _NS

_ZR = 80
_NZ = N // _ZR
_ZBASE = _NZ // _NS
_ZEXTRA = _NZ % _NS
_EPW = E // _NW
_CPW = _EPW // _K


@functools.cache
def _build_edge_sc():
    mesh = plsc.VectorSubcoreMesh(core_axis_name="c", subcore_axis_name="s")

    @functools.partial(
        pl.kernel,
        mesh=mesh,
        out_type=jax.ShapeDtypeStruct((_NC, N, H), jnp.float32),
        scratch_types=[
            pltpu.VMEM((_K,), jnp.int32),
            pltpu.VMEM((_K,), jnp.int32),
            pltpu.VMEM((_K, H), jnp.float32),
            pltpu.VMEM((_K, H), jnp.float32),
            pltpu.VMEM((_K, H), jnp.float32),
            pltpu.VMEM((_ZR, H), jnp.float32),
            pltpu.VMEM_SHARED((N, H), jnp.float32),
            pltpu.SemaphoreType.DMA,
            pltpu.SemaphoreType.DMA,
            pltpu.SemaphoreType.DMA,
        ],
    )
    def _edge_sc(a_hbm, b_hbm, c_hbm, src_hbm, dst_hbm, out_hbm,
                 src_v, dst_v, a_v, b_v, c_v, z_v, agg_sh, sa, sb, sc):
        cid = lax.axis_index("c")
        sid = lax.axis_index("s")
        wid = sid * _NC + cid

        def _zb(i, carry):
            for j in range(H // _LN):
                z_v[i, pl.ds(j * _LN, _LN)] = jnp.zeros((_LN,), jnp.float32)
            return carry

        lax.fori_loop(0, _ZR, _zb, 0)
        nz = _ZBASE + jnp.where(sid < _ZEXTRA, 1, 0)

        def _zfill(t, carry):
            pltpu.sync_copy(z_v, agg_sh.at[pl.ds((sid + t * _NS) * _ZR, _ZR)])
            return carry

        lax.fori_loop(0, nz, _zfill, 0)
        plsc.subcore_barrier()

        def _chunk(t, carry):
            base = wid * _EPW + t * _K
            pltpu.sync_copy(src_hbm.at[pl.ds(base, _K)], src_v)
            pltpu.sync_copy(dst_hbm.at[pl.ds(base, _K)], dst_v)
            ca = pltpu.async_copy(a_hbm.at[src_v], a_v, sa)
            cb = pltpu.async_copy(b_hbm.at[dst_v], b_v, sb)
            cc = pltpu.async_copy(c_hbm.at[pl.ds(base, _K)], c_v, sc)
            ca.wait()
            cb.wait()
            cc.wait()

            def _row(i, c2):
                for j in range(H // _LN):
                    s = pl.ds(j * _LN, _LN)
                    m = a_v[i, s] + b_v[i, s] + c_v[i, s]
                    a_v[i, s] = jnp.maximum(m, 0.0) + (
                        jnp.exp(jnp.minimum(m, 0.0)) - 1.0)
                return c2

            lax.fori_loop(0, _K, _row, 0)
            pltpu.sync_copy(a_v, agg_sh.at[dst_v], add=True)
            return carry

        lax.fori_loop(0, _CPW, _chunk, 0)
        plsc.subcore_barrier()

        def _dump(t, carry):
            off = (sid + t * _NS) * _ZR
            pltpu.sync_copy(agg_sh.at[pl.ds(off, _ZR)],
                            out_hbm.at[cid, pl.ds(off, _ZR)])
            return carry

        lax.fori_loop(0, nz, _dump, 0)

    return _edge_sc


def _edge_stage(A, B, C, src, dst):
    return _build_edge_sc()(A, B, C, src, dst)


def kernel(x, edge_index, edge_attr, batch, Wp, bp, Wm, bm, Wu, bu,
           W_ih, W_hh, b_ih, b_hh, W1, b1, ln_g, ln_b, W2, b2):
    src = edge_index[0]
    dst = edge_index[1]
    bp2 = bp.reshape(1, H)
    Wab = [jnp.concatenate([Wm[l, :H, :], Wm[l, H:2 * H, :]], axis=1)
           for l in range(L)]
    We = [Wm[l, 2 * H:, :] for l in range(L)]
    bm2 = [bm[l].reshape(1, H) for l in range(L)]
    bu2 = [bu[l].reshape(1, H) for l in range(L)]

    out, A, Bm = _proj_ab(x, Wp, bp2, Wab[0])
    for l in range(L):
        C = _edgeproj(edge_attr, We[l], bm2[l])
        P = _edge_stage(A, Bm, C, src, dst)
        if l + 1 < L:
            out, A, Bm = _upd_ab(out, P, Wu[l], bu2[l], Wab[l + 1])
        else:
            out = _upd(out, P, Wu[l], bu2[l])

    batch2d = batch.reshape(N, 1).astype(jnp.int32)
    return _s2s(out, batch2d, W_ih.T, W_hh.T,
                (b_ih + b_hh).reshape(1, 4 * H), W1, b1.reshape(1, H),
                ln_g.reshape(1, H), ln_b.reshape(1, H), W2, b2.reshape(1, H))
